# baseline (speedup 1.0000x reference)
import functools

import jax
import jax.numpy as jnp
from jax import lax
from jax.experimental import pallas as pl
from jax.experimental.pallas import tpu as pltpu
from jax.experimental.pallas import tpu_sc as plsc


def _embed_gather(n, dim, nc, ns):
    NW = nc * ns
    per_w = n // NW
    CHUNK = 512
    GPC = CHUNK // 128
    n_chunks = per_w // CHUNK
    rows_per_w = per_w // 128

    mesh = plsc.VectorSubcoreMesh(core_axis_name="c", subcore_axis_name="s")

    @functools.partial(
        pl.kernel,
        mesh=mesh,
        compiler_params=pltpu.CompilerParams(use_tc_tiling_on_sc=False),
        out_type=jax.ShapeDtypeStruct((n, dim), jnp.float32),
        scratch_types=[
            pltpu.VMEM((GPC, 128), jnp.int32),
            pltpu.VMEM((CHUNK, dim), jnp.float32),
            pltpu.SemaphoreType.DMA,
        ],
    )
    def k(idx_hbm, table_hbm, out_hbm, idx_v, rows_v, sem):
        wid = lax.axis_index("s") * nc + lax.axis_index("c")
        row_base = wid * rows_per_w
        out_base = wid * per_w

        @pl.loop(0, n_chunks)
        def _(g):
            pltpu.sync_copy(idx_hbm.at[pl.ds(row_base + g * GPC, GPC)], idx_v)
            copies = [
                pltpu.async_copy(
                    table_hbm.at[idx_v.at[j]],
                    rows_v.at[pl.ds(j * 128, 128)],
                    sem,
                )
                for j in range(GPC)
            ]
            for c in copies:
                c.wait()
            pltpu.sync_copy(rows_v, out_hbm.at[pl.ds(out_base + g * CHUNK, CHUNK)])

    return k


def kernel(x, table):
    B, H = x.shape
    V, D = table.shape
    n = B * H
    idx2d = x.reshape(n // 128, 128).astype(jnp.int32)
    info = plsc.get_sparse_core_info()
    out = _embed_gather(n, D, info.num_cores, info.num_subcores)(idx2d, table)
    return out.reshape(B, H, D)


# speedup vs baseline: 1.0492x; 1.0492x over previous
import functools

import jax
import jax.numpy as jnp
from jax import lax
from jax.experimental import pallas as pl
from jax.experimental.pallas import tpu as pltpu
from jax.experimental.pallas import tpu_sc as plsc

CHUNK = 256
IDXW = 128
GPC = CHUNK // IDXW
NBUF = 4


def _embed_gather(n, dim, nc, ns):
    NW = nc * ns
    per_w = n // NW
    n_chunks = per_w // CHUNK
    assert n_chunks % NBUF == 0
    idx_rows_per_w = per_w // IDXW

    mesh = plsc.VectorSubcoreMesh(core_axis_name="c", subcore_axis_name="s")

    @functools.partial(
        pl.kernel,
        mesh=mesh,
        compiler_params=pltpu.CompilerParams(use_tc_tiling_on_sc=False),
        out_type=jax.ShapeDtypeStruct((n, dim), jnp.float32),
        scratch_types=[
            pltpu.VMEM((NBUF, GPC, IDXW), jnp.int32),
            pltpu.VMEM((NBUF, CHUNK, dim), jnp.float32),
            pltpu.SemaphoreType.DMA,
            pltpu.SemaphoreType.DMA,
            pltpu.SemaphoreType.DMA,
        ],
    )
    def k(idx_hbm, table_hbm, out_hbm, idx_v, rows_v, sem_i, sem_g, sem_o):
        wid = lax.axis_index("s") * nc + lax.axis_index("c")
        row_base = wid * idx_rows_per_w
        out_base = wid * per_w

        def fire_idx(g, b):
            pltpu.async_copy(
                idx_hbm.at[pl.ds(row_base + g * GPC, GPC)], idx_v.at[b], sem_i
            )

        def wait_idx(b):
            pltpu.make_async_copy(
                idx_hbm.at[pl.ds(row_base, GPC)], idx_v.at[b], sem_i
            ).wait()

        def fire_gathers(g, b):
            for j in range(GPC):
                pltpu.async_copy(
                    table_hbm.at[idx_v.at[b].at[j]],
                    rows_v.at[b].at[pl.ds(j * IDXW, IDXW)],
                    sem_g,
                )

        def wait_gathers(b):
            pltpu.make_async_copy(
                out_hbm.at[pl.ds(out_base, CHUNK)], rows_v.at[b], sem_g
            ).wait()

        def fire_store(g, b):
            pltpu.async_copy(
                rows_v.at[b], out_hbm.at[pl.ds(out_base + g * CHUNK, CHUNK)], sem_o
            )

        def wait_store(b):
            pltpu.make_async_copy(
                rows_v.at[b], out_hbm.at[pl.ds(out_base, CHUNK)], sem_o
            ).wait()

        for b in range(NBUF):
            fire_idx(b, b)
        wait_idx(0)
        fire_gathers(0, 0)

        @pl.loop(0, n_chunks, step=NBUF)
        def _(g0):
            for b in range(NBUF):
                g = g0 + b
                b1 = (b + 1) % NBUF
                @pl.when(g + 1 < n_chunks)
                def _():
                    @pl.when(g + 1 >= NBUF)
                    def _():
                        wait_store(b1)
                    wait_idx(b1)
                    fire_gathers(g + 1, b1)
                wait_gathers(b)
                fire_store(g, b)
                @pl.when(g + NBUF < n_chunks)
                def _():
                    fire_idx(g + NBUF, b)

        for b in range(NBUF):
            wait_store(b)

    return k


def kernel(x, table):
    B, H = x.shape
    V, D = table.shape
    n = B * H
    idx2d = x.reshape(n // IDXW, IDXW).astype(jnp.int32)
    info = plsc.get_sparse_core_info()
    out = _embed_gather(n, D, info.num_cores, info.num_subcores)(idx2d, table)
    return out.reshape(B, H, D)
